# Initial kernel scaffold; baseline (speedup 1.0000x reference)
import jax
import jax.numpy as jnp
from jax.experimental import pallas as pl


def kernel(pos, pin_relate_x, pin_relate_y, num_vertices):
    raise NotImplementedError("write your pallas kernel here")


# speedup vs baseline: 238.9626x; 238.9626x over previous
import functools

import jax
import jax.numpy as jnp
from jax import lax
from jax.experimental import pallas as pl
from jax.experimental.pallas import tpu as pltpu
from jax.experimental.pallas import tpu_sc as plsc


def _gather_body(pos_hbm, px_hbm, py_hbm, outx_hbm, outy_hbm,
                 table_v, idx_v, out_v, *, num_pins, per_tile, ch):
    c = lax.axis_index("c")
    s = lax.axis_index("s")
    wid = s * 2 + c
    base = wid * per_tile
    nch = per_tile // ch

    for coord in range(2):
        idx_hbm = px_hbm if coord == 0 else py_hbm
        out_hbm = outx_hbm if coord == 0 else outy_hbm
        pltpu.sync_copy(pos_hbm.at[pl.ds(coord * num_pins, num_pins)], table_v)
        for k in range(nch):
            off = base + k * ch
            pltpu.sync_copy(idx_hbm.at[pl.ds(off, ch)], idx_v)

            def body(i, carry):
                idx = idx_v[pl.ds(i * 16, 16)]
                out_v[pl.ds(i * 16, 16)] = plsc.load_gather(table_v, [idx])
                return carry

            lax.fori_loop(0, ch // 16, body, 0)
            pltpu.sync_copy(out_v, out_hbm.at[pl.ds(off, ch)])


def kernel(pos, pin_relate_x, pin_relate_y, num_vertices):
    num_pins = pos.shape[0] // 2
    nv = pin_relate_x.shape[0]
    per_tile = nv // 32
    ch = 10000 if per_tile % 10000 == 0 else per_tile
    mesh = plsc.VectorSubcoreMesh(core_axis_name="c", subcore_axis_name="s")
    f = pl.kernel(
        functools.partial(_gather_body, num_pins=num_pins,
                          per_tile=per_tile, ch=ch),
        out_type=(jax.ShapeDtypeStruct((nv,), jnp.float32),
                  jax.ShapeDtypeStruct((nv,), jnp.float32)),
        mesh=mesh,
        scratch_types=[
            pltpu.VMEM((num_pins,), jnp.float32),
            pltpu.VMEM((ch,), jnp.int32),
            pltpu.VMEM((ch,), jnp.float32),
        ],
        compiler_params=pltpu.CompilerParams(needs_layout_passes=False),
    )
    return f(pos, pin_relate_x, pin_relate_y)


# speedup vs baseline: 312.9902x; 1.3098x over previous
import functools

import jax
import jax.numpy as jnp
from jax import lax
from jax.experimental import pallas as pl
from jax.experimental.pallas import tpu as pltpu
from jax.experimental.pallas import tpu_sc as plsc

_CH = 2000


def _gather_body(pos_hbm, px_hbm, py_hbm, outx_hbm, outy_hbm,
                 table_v, idx0, idx1, out0, out1,
                 tsem, isem0, isem1, osem0, osem1,
                 *, num_pins, per_tile):
    c = lax.axis_index("c")
    s = lax.axis_index("s")
    wid = s * 2 + c
    base = wid * per_tile
    nch = per_tile // _CH
    bufs = [(idx0, isem0, out0, osem0), (idx1, isem1, out1, osem1)]

    for coord in range(2):
        idx_hbm = px_hbm if coord == 0 else py_hbm
        out_hbm = outx_hbm if coord == 0 else outy_hbm

        tcopy = pltpu.async_copy(
            pos_hbm.at[pl.ds(coord * num_pins, num_pins)], table_v, tsem)
        first = pltpu.async_copy(
            idx_hbm.at[pl.ds(base, _CH)], bufs[0][0], bufs[0][1])
        tcopy.wait()

        in_flight = {0: first}
        out_flight = {}
        for k in range(nch):
            idx_v, isem, out_v, osem = bufs[k % 2]
            if k + 1 < nch:
                nidx, nisem, _, _ = bufs[(k + 1) % 2]
                in_flight[k + 1] = pltpu.async_copy(
                    idx_hbm.at[pl.ds(base + (k + 1) * _CH, _CH)], nidx, nisem)
            in_flight.pop(k).wait()
            if k - 2 in out_flight:
                out_flight.pop(k - 2).wait()

            @plsc.parallel_loop(0, _CH // 16, 1, unroll=8)
            def _(i):
                idx = idx_v[pl.ds(i * 16, 16)]
                out_v[pl.ds(i * 16, 16)] = plsc.load_gather(table_v, [idx])

            out_flight[k] = pltpu.async_copy(
                out_v, out_hbm.at[pl.ds(base + k * _CH, _CH)], osem)
        for k in sorted(out_flight):
            out_flight.pop(k).wait()


def kernel(pos, pin_relate_x, pin_relate_y, num_vertices):
    num_pins = pos.shape[0] // 2
    nv = pin_relate_x.shape[0]
    per_tile = nv // 32
    mesh = plsc.VectorSubcoreMesh(core_axis_name="c", subcore_axis_name="s")
    f = pl.kernel(
        functools.partial(_gather_body, num_pins=num_pins, per_tile=per_tile),
        out_type=(jax.ShapeDtypeStruct((nv,), jnp.float32),
                  jax.ShapeDtypeStruct((nv,), jnp.float32)),
        mesh=mesh,
        scratch_types=[
            pltpu.VMEM((num_pins,), jnp.float32),
            pltpu.VMEM((_CH,), jnp.int32),
            pltpu.VMEM((_CH,), jnp.int32),
            pltpu.VMEM((_CH,), jnp.float32),
            pltpu.VMEM((_CH,), jnp.float32),
            pltpu.SemaphoreType.DMA,
            pltpu.SemaphoreType.DMA,
            pltpu.SemaphoreType.DMA,
            pltpu.SemaphoreType.DMA,
            pltpu.SemaphoreType.DMA,
        ],
        compiler_params=pltpu.CompilerParams(needs_layout_passes=False),
    )
    return f(pos, pin_relate_x, pin_relate_y)


# speedup vs baseline: 375.5604x; 1.1999x over previous
import functools

import jax
import jax.numpy as jnp
from jax import lax
from jax.experimental import pallas as pl
from jax.experimental.pallas import tpu as pltpu
from jax.experimental.pallas import tpu_sc as plsc

_CH = 2000


def _gather_body(pos_hbm, px_hbm, py_hbm, outx_hbm, outy_hbm,
                 table_v, idx0, idx1, out0, out1, spmx, spmy,
                 tsemx, tsemy, isem0, isem1, osem0, osem1,
                 *, num_pins, per_tile):
    c = lax.axis_index("c")
    s = lax.axis_index("s")
    wid = s * 2 + c
    base = wid * per_tile
    nch = per_tile // _CH
    bufs = [(idx0, isem0, out0, osem0), (idx1, isem1, out1, osem1)]

    xstage = pltpu.make_async_copy(
        pos_hbm.at[pl.ds(0, num_pins)], spmx, tsemx)
    ystage = pltpu.make_async_copy(
        pos_hbm.at[pl.ds(num_pins, num_pins)], spmy, tsemy)

    @pl.when(s == 0)
    def _():
        xstage.start()
        ystage.start()
        xstage.wait()

    plsc.subcore_barrier()

    for coord in range(2):
        idx_hbm = px_hbm if coord == 0 else py_hbm
        out_hbm = outx_hbm if coord == 0 else outy_hbm
        spm = spmx if coord == 0 else spmy

        first = pltpu.async_copy(
            idx_hbm.at[pl.ds(base, _CH)], bufs[0][0], bufs[0][1])
        pltpu.sync_copy(spm, table_v)

        in_flight = {0: first}
        out_flight = {}
        for k in range(nch):
            idx_v, isem, out_v, osem = bufs[k % 2]
            if k + 1 < nch:
                nidx, nisem, _, _ = bufs[(k + 1) % 2]
                in_flight[k + 1] = pltpu.async_copy(
                    idx_hbm.at[pl.ds(base + (k + 1) * _CH, _CH)], nidx, nisem)
            in_flight.pop(k).wait()
            if k - 2 in out_flight:
                out_flight.pop(k - 2).wait()

            @plsc.parallel_loop(0, _CH // 16, 1, unroll=8)
            def _(i):
                idx = idx_v[pl.ds(i * 16, 16)]
                out_v[pl.ds(i * 16, 16)] = plsc.load_gather(table_v, [idx])

            out_flight[k] = pltpu.async_copy(
                out_v, out_hbm.at[pl.ds(base + k * _CH, _CH)], osem)
        for k in sorted(out_flight):
            out_flight.pop(k).wait()

        if coord == 0:
            @pl.when(s == 0)
            def _():
                ystage.wait()

            plsc.subcore_barrier()


def kernel(pos, pin_relate_x, pin_relate_y, num_vertices):
    num_pins = pos.shape[0] // 2
    nv = pin_relate_x.shape[0]
    per_tile = nv // 32
    mesh = plsc.VectorSubcoreMesh(core_axis_name="c", subcore_axis_name="s")
    f = pl.kernel(
        functools.partial(_gather_body, num_pins=num_pins, per_tile=per_tile),
        out_type=(jax.ShapeDtypeStruct((nv,), jnp.float32),
                  jax.ShapeDtypeStruct((nv,), jnp.float32)),
        mesh=mesh,
        scratch_types=[
            pltpu.VMEM((num_pins,), jnp.float32),
            pltpu.VMEM((_CH,), jnp.int32),
            pltpu.VMEM((_CH,), jnp.int32),
            pltpu.VMEM((_CH,), jnp.float32),
            pltpu.VMEM((_CH,), jnp.float32),
            pltpu.VMEM_SHARED((num_pins,), jnp.float32),
            pltpu.VMEM_SHARED((num_pins,), jnp.float32),
            pltpu.SemaphoreType.DMA,
            pltpu.SemaphoreType.DMA,
            pltpu.SemaphoreType.DMA,
            pltpu.SemaphoreType.DMA,
            pltpu.SemaphoreType.DMA,
            pltpu.SemaphoreType.DMA,
        ],
        compiler_params=pltpu.CompilerParams(
            needs_layout_passes=False, use_tc_tiling_on_sc=False),
    )
    return f(pos, pin_relate_x, pin_relate_y)


# speedup vs baseline: 480.4131x; 1.2792x over previous
import functools

import jax
import jax.numpy as jnp
from jax import lax
from jax.experimental import pallas as pl
from jax.experimental.pallas import tpu as pltpu
from jax.experimental.pallas import tpu_sc as plsc

_CH = 2000
_IDX_D = 6
_OUT_D = 3


def _phase(idx_hbm, out_hbm, spm, table_v, ibufs, isems, obufs, osems,
           base, nch):
    in_flight = {}
    for k in range(min(_IDX_D, nch)):
        in_flight[k] = pltpu.async_copy(
            idx_hbm.at[pl.ds(base + k * _CH, _CH)], ibufs[k], isems[k])

    pltpu.sync_copy(spm, table_v)

    out_flight = {}
    for k in range(nch):
        idx_v, isem = ibufs[k % _IDX_D], isems[k % _IDX_D]
        out_v, osem = obufs[k % _OUT_D], osems[k % _OUT_D]
        in_flight.pop(k).wait()
        if k - _OUT_D in out_flight:
            out_flight.pop(k - _OUT_D).wait()

        @plsc.parallel_loop(0, _CH // 16, 1, unroll=8)
        def _(i):
            idx = idx_v[pl.ds(i * 16, 16)]
            out_v[pl.ds(i * 16, 16)] = plsc.load_gather(table_v, [idx])

        out_flight[k] = pltpu.async_copy(
            out_v, out_hbm.at[pl.ds(base + k * _CH, _CH)], osem)
        if k + _IDX_D < nch:
            in_flight[k + _IDX_D] = pltpu.async_copy(
                idx_hbm.at[pl.ds(base + (k + _IDX_D) * _CH, _CH)],
                idx_v, isem)
    for k in sorted(out_flight):
        out_flight.pop(k).wait()


def _gather_body(pos_hbm, px_hbm, py_hbm, outx_hbm, outy_hbm,
                 table_v, spm,
                 i0, i1, i2, i3, i4, i5, o0, o1, o2,
                 tsem, is0, is1, is2, is3, is4, is5, os0, os1, os2,
                 *, num_pins, per_tile):
    c = lax.axis_index("c")
    s = lax.axis_index("s")
    base = s * per_tile
    nch = per_tile // _CH
    ibufs = [i0, i1, i2, i3, i4, i5]
    isems = [is0, is1, is2, is3, is4, is5]
    obufs = [o0, o1, o2]
    osems = [os0, os1, os2]

    @pl.when(c == 0)
    def _():
        st = pltpu.make_async_copy(pos_hbm.at[pl.ds(0, num_pins)], spm, tsem)

        @pl.when(s == 0)
        def _():
            st.start()
            st.wait()

        plsc.subcore_barrier()
        _phase(px_hbm, outx_hbm, spm, table_v, ibufs, isems, obufs, osems,
               base, nch)

    @pl.when(c == 1)
    def _():
        st = pltpu.make_async_copy(
            pos_hbm.at[pl.ds(num_pins, num_pins)], spm, tsem)

        @pl.when(s == 0)
        def _():
            st.start()
            st.wait()

        plsc.subcore_barrier()
        _phase(py_hbm, outy_hbm, spm, table_v, ibufs, isems, obufs, osems,
               base, nch)


def kernel(pos, pin_relate_x, pin_relate_y, num_vertices):
    num_pins = pos.shape[0] // 2
    nv = pin_relate_x.shape[0]
    per_tile = nv // 16
    mesh = plsc.VectorSubcoreMesh(core_axis_name="c", subcore_axis_name="s")
    f = pl.kernel(
        functools.partial(_gather_body, num_pins=num_pins, per_tile=per_tile),
        out_type=(jax.ShapeDtypeStruct((nv,), jnp.float32),
                  jax.ShapeDtypeStruct((nv,), jnp.float32)),
        mesh=mesh,
        scratch_types=(
            [pltpu.VMEM((num_pins,), jnp.float32),
             pltpu.VMEM_SHARED((num_pins,), jnp.float32)]
            + [pltpu.VMEM((_CH,), jnp.int32) for _ in range(_IDX_D)]
            + [pltpu.VMEM((_CH,), jnp.float32) for _ in range(_OUT_D)]
            + [pltpu.SemaphoreType.DMA for _ in range(1 + _IDX_D + _OUT_D)]
        ),
        compiler_params=pltpu.CompilerParams(
            needs_layout_passes=False, use_tc_tiling_on_sc=False),
    )
    return f(pos, pin_relate_x, pin_relate_y)


# speedup vs baseline: 557.8278x; 1.1611x over previous
import functools

import jax
import jax.numpy as jnp
from jax import lax
from jax.experimental import pallas as pl
from jax.experimental.pallas import tpu as pltpu
from jax.experimental.pallas import tpu_sc as plsc

_CH = 2000
_RING = 5


def _phase(pos_hbm, idx_hbm, out_hbm, spm, table_v, ibufs, isems,
           obufs, osems, psema, psemb, base, half_off, num_pins, ncyc):
    half = num_pins // 2

    for t in range(_RING):
        pltpu.async_copy(
            idx_hbm.at[pl.ds(base + t * _CH, _CH)], ibufs[t], isems[t])

    pull_a = pltpu.async_copy(spm, table_v.at[pl.ds(0, half)], psema)
    pull_b = pltpu.async_copy(
        pos_hbm.at[pl.ds(half_off + half, half)],
        table_v.at[pl.ds(half, half)], psemb)
    pull_a.wait()
    pull_b.wait()

    @pl.loop(0, ncyc)
    def _(j):
        jbase = base + j * (_RING * _CH)
        for t in range(_RING):
            off = jbase + t * _CH
            pltpu.make_async_copy(
                idx_hbm.at[pl.ds(off, _CH)], ibufs[t], isems[t]).wait()

            @pl.when(j > 0)
            def _():
                pltpu.make_async_copy(
                    obufs[t], out_hbm.at[pl.ds(off, _CH)], osems[t]).wait()

            idx_v, out_v = ibufs[t], obufs[t]

            @plsc.parallel_loop(0, _CH // 16, 1, unroll=8)
            def _(i):
                idx = idx_v[pl.ds(i * 16, 16)]
                out_v[pl.ds(i * 16, 16)] = plsc.load_gather(table_v, [idx])

            pltpu.async_copy(out_v, out_hbm.at[pl.ds(off, _CH)], osems[t])

            @pl.when(j + 1 < ncyc)
            def _():
                pltpu.async_copy(
                    idx_hbm.at[pl.ds(off + _RING * _CH, _CH)],
                    ibufs[t], isems[t])

    for t in range(_RING):
        pltpu.make_async_copy(
            obufs[t], out_hbm.at[pl.ds(0, _CH)], osems[t]).wait()


def _gather_body(pos_hbm, px_hbm, py_hbm, outx_hbm, outy_hbm,
                 table_v, spm,
                 i0, i1, i2, i3, i4, o0, o1, o2, o3, o4,
                 tsem, pa, pb, is0, is1, is2, is3, is4,
                 os0, os1, os2, os3, os4,
                 *, num_pins, per_tile):
    c = lax.axis_index("c")
    s = lax.axis_index("s")
    base = s * per_tile
    ncyc = per_tile // (_RING * _CH)
    ibufs = [i0, i1, i2, i3, i4]
    isems = [is0, is1, is2, is3, is4]
    obufs = [o0, o1, o2, o3, o4]
    osems = [os0, os1, os2, os3, os4]
    half = num_pins // 2

    @pl.when(c == 0)
    def _():
        st = pltpu.make_async_copy(pos_hbm.at[pl.ds(0, half)], spm, tsem)

        @pl.when(s == 0)
        def _():
            st.start()
            st.wait()

        plsc.subcore_barrier()
        _phase(pos_hbm, px_hbm, outx_hbm, spm, table_v, ibufs, isems,
               obufs, osems, pa, pb, base, 0, num_pins, ncyc)

    @pl.when(c == 1)
    def _():
        st = pltpu.make_async_copy(
            pos_hbm.at[pl.ds(num_pins, half)], spm, tsem)

        @pl.when(s == 0)
        def _():
            st.start()
            st.wait()

        plsc.subcore_barrier()
        _phase(pos_hbm, py_hbm, outy_hbm, spm, table_v, ibufs, isems,
               obufs, osems, pa, pb, base, num_pins, num_pins, ncyc)


def kernel(pos, pin_relate_x, pin_relate_y, num_vertices):
    num_pins = pos.shape[0] // 2
    nv = pin_relate_x.shape[0]
    per_tile = nv // 16
    mesh = plsc.VectorSubcoreMesh(core_axis_name="c", subcore_axis_name="s")
    f = pl.kernel(
        functools.partial(_gather_body, num_pins=num_pins, per_tile=per_tile),
        out_type=(jax.ShapeDtypeStruct((nv,), jnp.float32),
                  jax.ShapeDtypeStruct((nv,), jnp.float32)),
        mesh=mesh,
        scratch_types=(
            [pltpu.VMEM((num_pins,), jnp.float32),
             pltpu.VMEM_SHARED((num_pins // 2,), jnp.float32)]
            + [pltpu.VMEM((_CH,), jnp.int32) for _ in range(_RING)]
            + [pltpu.VMEM((_CH,), jnp.float32) for _ in range(_RING)]
            + [pltpu.SemaphoreType.DMA for _ in range(3 + 2 * _RING)]
        ),
        compiler_params=pltpu.CompilerParams(
            needs_layout_passes=False, use_tc_tiling_on_sc=False),
    )
    return f(pos, pin_relate_x, pin_relate_y)
